# Initial kernel scaffold; baseline (speedup 1.0000x reference)
import jax
import jax.numpy as jnp
from jax.experimental import pallas as pl


def kernel(x, y, batch, params):
    raise NotImplementedError("write your pallas kernel here")


# speedup vs baseline: 2.6879x; 2.6879x over previous
import functools

import jax
import jax.numpy as jnp
from jax import lax
from jax.experimental import pallas as pl
from jax.experimental.pallas import tpu as pltpu

G = 100
P = 100
PP = 104
K = 16
N = G * P
E_REAL = float(N * K)
BIG = 1e30
EPS = 1e-5

_INTERPRET = False


def _pc(body, grid, in_specs, out_specs, out_shape, scratch_shapes=()):
    return pl.pallas_call(
        body,
        grid=grid,
        in_specs=in_specs,
        out_specs=out_specs,
        out_shape=out_shape,
        scratch_shapes=list(scratch_shapes),
        interpret=_INTERPRET,
    )


def _knn_body(pos_ref, *out_refs, d_pos, pre):
    pb = pos_ref[0]
    if pre:
        col = lax.broadcasted_iota(jnp.int32, pb.shape, 1)
        shift = jnp.where(col == 2, -1.7, 0.0)
        shift = jnp.where(col == 3, -2.0, shift)
        shift = jnp.where((col == 4) | (col == 5), 4.7, shift)
        shift = jnp.where(col == 6, -0.2, shift)
        mult = jnp.where((col >= 2) & (col <= 5), 0.7, 1.0)
        mult = jnp.where(col == 6, 4.7, mult)
        feats = (pb + shift) * mult
        out_refs[1][0] = feats
        pos = pb[:, :d_pos]
    else:
        pos = pb[:, :d_pos]
    nsq = jnp.sum(pos * pos, axis=1, keepdims=True)
    ones = jnp.full((PP, 1), 1.0, jnp.float32)
    u = jnp.concatenate([-2.0 * pos, ones], axis=1)
    v = jnp.concatenate([pos, nsq], axis=1)
    dist = lax.dot_general(u, v, (((1,), (1,)), ((), ())),
                           preferred_element_type=jnp.float32)
    lane = lax.broadcasted_iota(jnp.int32, (PP, PP), 1)
    row = lax.broadcasted_iota(jnp.int32, (PP, PP), 0)
    dist = jnp.where((lane == row) | (lane >= P), BIG, dist)
    idx_ref = out_refs[0]
    cols = []
    for _ in range(K):
        m = jnp.min(dist, axis=1, keepdims=True)
        idx = jnp.min(jnp.where(dist <= m, lane, PP * 4), axis=1,
                      keepdims=True)
        cols.append(idx)
        dist = jnp.where(lane == idx, BIG, dist)
    idx_ref[0] = jnp.concatenate(cols, axis=1)


def _knn(feats_p, d_pos, pre=False):
    d = feats_p.shape[-1]
    out_shape = [jax.ShapeDtypeStruct((G, PP, K), jnp.int32)]
    out_specs = [pl.BlockSpec((1, PP, K), lambda g: (g, 0, 0))]
    if pre:
        out_shape.append(jax.ShapeDtypeStruct((G, PP, d), jnp.float32))
        out_specs.append(pl.BlockSpec((1, PP, d), lambda g: (g, 0, 0)))
    res = _pc(
        functools.partial(_knn_body, d_pos=d_pos, pre=pre),
        grid=(G,),
        in_specs=[pl.BlockSpec((1, PP, d), lambda g: (g, 0, 0))],
        out_specs=out_specs if pre else out_specs[0],
        out_shape=out_shape if pre else out_shape[0],
    )(feats_p)
    return res


def _nodelin_body(f_ref, w_ref, y_ref):
    y_ref[...] = jnp.dot(f_ref[...], w_ref[...],
                         preferred_element_type=jnp.float32)


def _nodelin(feats_p, w_cat):
    d = feats_p.shape[-1]
    c2 = w_cat.shape[-1]
    y = _pc(
        _nodelin_body,
        grid=(1,),
        in_specs=[pl.BlockSpec((G * PP, d), lambda i: (0, 0)),
                  pl.BlockSpec((d, c2), lambda i: (0, 0))],
        out_specs=pl.BlockSpec((G * PP, c2), lambda i: (0, 0)),
        out_shape=jax.ShapeDtypeStruct((G * PP, c2), jnp.float32),
    )(feats_p.reshape(G * PP, d), w_cat)
    return y.reshape(G, PP, c2)


def _edge1_body(y_ref, src_ref, b_ref, h_ref, st_ref, o_scr, acc, *, c):
    g = pl.program_id(0)
    yb = y_ref[0]
    a = yb[:, :c]
    bmat = yb[:, c:]
    src = src_ref[0]
    lane = lax.broadcasted_iota(jnp.int32, (PP, PP), 1)
    for k in range(K):
        sk = src[:, k:k + 1]
        o_scr[k * PP:(k + 1) * PP, :] = (lane == sk).astype(jnp.float32)
    hj = jnp.dot(o_scr[...], bmat, preferred_element_type=jnp.float32)
    h = hj.reshape(K, PP, c) + a[None] + b_ref[...][None]
    h_ref[0] = h
    real = lax.broadcasted_iota(jnp.int32, (K, PP, 1), 1) < P
    hm = jnp.where(real, h, 0.0)
    s = jnp.sum(hm, axis=(0, 1))[None]
    ss = jnp.sum(jnp.where(real, h * h, 0.0), axis=(0, 1))[None]

    @pl.when(g == 0)
    def _():
        acc[...] = jnp.zeros_like(acc)

    acc[0:1, :] += s
    acc[1:2, :] += ss

    @pl.when(g == G - 1)
    def _():
        st_ref[...] = acc[...]


def _edge1(y, src, b1, c):
    return _pc(
        functools.partial(_edge1_body, c=c),
        grid=(G,),
        in_specs=[pl.BlockSpec((1, PP, 2 * c), lambda g: (g, 0, 0)),
                  pl.BlockSpec((1, PP, K), lambda g: (g, 0, 0)),
                  pl.BlockSpec((1, c), lambda g: (0, 0))],
        out_specs=[pl.BlockSpec((1, K, PP, c), lambda g: (g, 0, 0, 0)),
                   pl.BlockSpec((8, c), lambda g: (0, 0))],
        out_shape=[jax.ShapeDtypeStruct((G, K, PP, c), jnp.float32),
                   jax.ShapeDtypeStruct((8, c), jnp.float32)],
        scratch_shapes=[pltpu.VMEM((K * PP, PP), jnp.float32),
                        pltpu.VMEM((8, c), jnp.float32)],
    )(y, src, b1)


def _mid_body(h_ref, st_ref, g_ref, be_ref, w_ref, b_ref,
              ho_ref, sto_ref, acc, *, c, gb, nsteps):
    t = pl.program_id(0)
    r = gb * K * PP
    hp = h_ref[...].reshape(r, c)
    mu = st_ref[0:1, :] / E_REAL
    var = st_ref[1:2, :] / E_REAL - mu * mu
    scale = g_ref[...] * lax.rsqrt(var + EPS)
    shift = be_ref[...] - mu * scale
    xn = jnp.maximum(hp * scale + shift, 0.0)
    hn = jnp.dot(xn, w_ref[...], preferred_element_type=jnp.float32) + b_ref[...]
    ho_ref[...] = hn.reshape(gb, K, PP, c)
    node = lax.broadcasted_iota(jnp.int32, (r, 1), 0) % PP
    real = node < P
    s = jnp.sum(jnp.where(real, hn, 0.0), axis=0)[None]
    ss = jnp.sum(jnp.where(real, hn * hn, 0.0), axis=0)[None]

    @pl.when(t == 0)
    def _():
        acc[...] = jnp.zeros_like(acc)

    acc[0:1, :] += s
    acc[1:2, :] += ss

    @pl.when(t == nsteps - 1)
    def _():
        sto_ref[...] = acc[...]


def _mid(h, st, gamma, beta, w, b, c, gb=5):
    nsteps = G // gb
    return _pc(
        functools.partial(_mid_body, c=c, gb=gb, nsteps=nsteps),
        grid=(nsteps,),
        in_specs=[pl.BlockSpec((gb, K, PP, c), lambda t: (t, 0, 0, 0)),
                  pl.BlockSpec((8, c), lambda t: (0, 0)),
                  pl.BlockSpec((1, c), lambda t: (0, 0)),
                  pl.BlockSpec((1, c), lambda t: (0, 0)),
                  pl.BlockSpec((c, c), lambda t: (0, 0)),
                  pl.BlockSpec((1, c), lambda t: (0, 0))],
        out_specs=[pl.BlockSpec((gb, K, PP, c), lambda t: (t, 0, 0, 0)),
                   pl.BlockSpec((8, c), lambda t: (0, 0))],
        out_shape=[jax.ShapeDtypeStruct((G, K, PP, c), jnp.float32),
                   jax.ShapeDtypeStruct((8, c), jnp.float32)],
        scratch_shapes=[pltpu.VMEM((8, c), jnp.float32)],
    )(h, st, gamma, beta, w, b)


def _agg_body(h_ref, st_ref, g_ref, be_ref, o_ref, *, c, gb):
    mu = st_ref[0:1, :] / E_REAL
    var = st_ref[1:2, :] / E_REAL - mu * mu
    scale = g_ref[...] * lax.rsqrt(var + EPS)
    shift = be_ref[...] - mu * scale
    h = h_ref[...]
    r = jnp.maximum(h * scale[None, None] + shift[None, None], 0.0)
    o_ref[...] = jnp.sum(r, axis=1) * (1.0 / K)


def _agg(h, st, gamma, beta, c, gb=5):
    nsteps = G // gb
    return _pc(
        functools.partial(_agg_body, c=c, gb=gb),
        grid=(nsteps,),
        in_specs=[pl.BlockSpec((gb, K, PP, c), lambda t: (t, 0, 0, 0)),
                  pl.BlockSpec((8, c), lambda t: (0, 0)),
                  pl.BlockSpec((1, c), lambda t: (0, 0)),
                  pl.BlockSpec((1, c), lambda t: (0, 0))],
        out_specs=pl.BlockSpec((gb, PP, c), lambda t: (t, 0, 0)),
        out_shape=jax.ShapeDtypeStruct((G, PP, c), jnp.float32),
    )(h, st, gamma, beta)


def _head_body(f_ref, w1_ref, b1_ref, w2_ref, b2_ref, o_ref, *, d):
    f = f_ref[...]
    real = lax.broadcasted_iota(jnp.int32, (1, PP, 1), 1) < P
    pooled = jnp.sum(jnp.where(real, f, 0.0), axis=1) * (1.0 / P)
    h = jnp.maximum(jnp.dot(pooled, w1_ref[...],
                            preferred_element_type=jnp.float32) + b1_ref[...],
                    0.0)
    z = jnp.sum(h * w2_ref[...], axis=1, keepdims=True) + b2_ref[...]
    o_ref[...] = 1.0 / (1.0 + jnp.exp(-z))


def _head(feats_p, w1, b1, w2, b2):
    d = feats_p.shape[-1]
    return _pc(
        functools.partial(_head_body, d=d),
        grid=(1,),
        in_specs=[pl.BlockSpec((G, PP, d), lambda i: (0, 0, 0)),
                  pl.BlockSpec((d, 256), lambda i: (0, 0)),
                  pl.BlockSpec((1, 256), lambda i: (0, 0)),
                  pl.BlockSpec((1, 256), lambda i: (0, 0)),
                  pl.BlockSpec((1, 1), lambda i: (0, 0))],
        out_specs=pl.BlockSpec((G, 1), lambda i: (0, 0)),
        out_shape=jax.ShapeDtypeStruct((G, 1), jnp.float32),
    )(feats_p, w1, b1, w2, b2)


def kernel(x, y, batch, params):
    xp = jnp.pad(x.reshape(G, P, -1), ((0, 0), (0, PP - P), (0, 0)))
    feats = None
    pos_dims = [2, 71, 199]
    for i in range(3):
        c = [64, 128, 256][i]
        layers = params["blocks"][i]
        if i == 0:
            src, feats = _knn(xp, pos_dims[0], pre=True)
        else:
            src = _knn(feats, pos_dims[i])
        d = feats.shape[-1]
        w0, b0, g0, be0 = layers[0]
        y1 = _nodelin(feats, jnp.concatenate([w0[:d], w0[d:]], axis=1))
        h, st = _edge1(y1, src, b0.reshape(1, c), c)
        for j in (1, 2):
            wj, bj, gj, bej = layers[j]
            gprev = layers[j - 1][2].reshape(1, c)
            beprev = layers[j - 1][3].reshape(1, c)
            h, st = _mid(h, st, gprev, beprev, wj, bj.reshape(1, c), c)
        out = _agg(h, st, layers[2][2].reshape(1, c), layers[2][3].reshape(1, c), c)
        feats = jnp.concatenate([out, feats], axis=2)
    pred = _head(feats, params["W1"], params["b1"].reshape(1, 256),
                 params["W2"].reshape(1, 256), params["b2"].reshape(1, 1))
    return (pred, y)


# speedup vs baseline: 4.8315x; 1.7975x over previous
import functools

import jax
import jax.numpy as jnp
from jax import lax
from jax.experimental import pallas as pl
from jax.experimental.pallas import tpu as pltpu

G = 100
P = 100
PP = 104
K = 16
N = G * P
E_REAL = float(N * K)
BIG = 1e30
EPS = 1e-5

_INTERPRET = False


def _pc(body, grid, in_specs, out_specs, out_shape, scratch_shapes=()):
    return pl.pallas_call(
        body,
        grid=grid,
        in_specs=in_specs,
        out_specs=out_specs,
        out_shape=out_shape,
        scratch_shapes=list(scratch_shapes),
        interpret=_INTERPRET,
    )


GBK = 20


def _knn_body(pos_ref, *out_refs, d_pos, pre):
    pb = pos_ref[...]
    if pre:
        col = lax.broadcasted_iota(jnp.int32, pb.shape, 2)
        shift = jnp.where(col == 2, -1.7, 0.0)
        shift = jnp.where(col == 3, -2.0, shift)
        shift = jnp.where((col == 4) | (col == 5), 4.7, shift)
        shift = jnp.where(col == 6, -0.2, shift)
        mult = jnp.where((col >= 2) & (col <= 5), 0.7, 1.0)
        mult = jnp.where(col == 6, 4.7, mult)
        out_refs[1][...] = (pb + shift) * mult
    mats = []
    ones = jnp.full((PP, 1), 1.0, jnp.float32)
    for gg in range(GBK):
        pos = pb[gg, :, :d_pos]
        nsq = jnp.sum(pos * pos, axis=1, keepdims=True)
        u = jnp.concatenate([-2.0 * pos, ones], axis=1)
        v = jnp.concatenate([pos, nsq], axis=1)
        mats.append(lax.dot_general(u, v, (((1,), (1,)), ((), ())),
                                    preferred_element_type=jnp.float32))
    dist = jnp.concatenate(mats, axis=0)
    r = GBK * PP
    lane = lax.broadcasted_iota(jnp.int32, (r, PP), 1)
    row = lax.broadcasted_iota(jnp.int32, (r, PP), 0) % PP
    dist = jnp.where((lane == row) | (lane >= P), BIG, dist)
    cols = []
    for _ in range(K):
        m = jnp.min(dist, axis=1, keepdims=True)
        idx = jnp.min(jnp.where(dist <= m, lane, PP * 4), axis=1,
                      keepdims=True)
        cols.append(idx)
        dist = jnp.where(lane == idx, BIG, dist)
    out_refs[0][...] = jnp.concatenate(cols, axis=1).reshape(GBK, PP, K)


def _knn(feats_p, d_pos, pre=False):
    d = feats_p.shape[-1]
    out_shape = [jax.ShapeDtypeStruct((G, PP, K), jnp.int32)]
    out_specs = [pl.BlockSpec((GBK, PP, K), lambda g: (g, 0, 0))]
    if pre:
        out_shape.append(jax.ShapeDtypeStruct((G, PP, d), jnp.float32))
        out_specs.append(pl.BlockSpec((GBK, PP, d), lambda g: (g, 0, 0)))
    res = _pc(
        functools.partial(_knn_body, d_pos=d_pos, pre=pre),
        grid=(G // GBK,),
        in_specs=[pl.BlockSpec((GBK, PP, d), lambda g: (g, 0, 0))],
        out_specs=out_specs if pre else out_specs[0],
        out_shape=out_shape if pre else out_shape[0],
    )(feats_p)
    return res


def _nodelin_body(f_ref, w_ref, y_ref):
    y_ref[...] = jnp.dot(f_ref[...], w_ref[...],
                         preferred_element_type=jnp.float32)


def _nodelin(feats_p, w_cat):
    d = feats_p.shape[-1]
    c2 = w_cat.shape[-1]
    y = _pc(
        _nodelin_body,
        grid=(1,),
        in_specs=[pl.BlockSpec((G * PP, d), lambda i: (0, 0)),
                  pl.BlockSpec((d, c2), lambda i: (0, 0))],
        out_specs=pl.BlockSpec((G * PP, c2), lambda i: (0, 0)),
        out_shape=jax.ShapeDtypeStruct((G * PP, c2), jnp.float32),
    )(feats_p.reshape(G * PP, d), w_cat)
    return y.reshape(G, PP, c2)


def _edge1_body(y_ref, src_ref, b_ref, h_ref, st_ref, o_scr, acc, *, c):
    g = pl.program_id(0)
    yb = y_ref[0]
    a = yb[:, :c]
    bmat = yb[:, c:]
    src = src_ref[0]
    lane = lax.broadcasted_iota(jnp.int32, (PP, PP), 1)
    for k in range(K):
        sk = src[:, k:k + 1]
        o_scr[k * PP:(k + 1) * PP, :] = (lane == sk).astype(jnp.float32)
    hj = jnp.dot(o_scr[...], bmat, preferred_element_type=jnp.float32)
    h = hj.reshape(K, PP, c) + a[None] + b_ref[...][None]
    h_ref[0] = h
    hf = h.reshape(K * PP, c)
    mrow = (lax.broadcasted_iota(jnp.int32, (1, K * PP), 1) % PP
            < P).astype(jnp.float32)
    s = jnp.dot(mrow, hf, preferred_element_type=jnp.float32)
    ss = jnp.dot(mrow, hf * hf, preferred_element_type=jnp.float32)

    @pl.when(g == 0)
    def _():
        acc[...] = jnp.zeros_like(acc)

    acc[0:1, :] += s
    acc[1:2, :] += ss

    @pl.when(g == G - 1)
    def _():
        st_ref[...] = acc[...]


def _edge1(y, src, b1, c):
    return _pc(
        functools.partial(_edge1_body, c=c),
        grid=(G,),
        in_specs=[pl.BlockSpec((1, PP, 2 * c), lambda g: (g, 0, 0)),
                  pl.BlockSpec((1, PP, K), lambda g: (g, 0, 0)),
                  pl.BlockSpec((1, c), lambda g: (0, 0))],
        out_specs=[pl.BlockSpec((1, K, PP, c), lambda g: (g, 0, 0, 0)),
                   pl.BlockSpec((8, c), lambda g: (0, 0))],
        out_shape=[jax.ShapeDtypeStruct((G, K, PP, c), jnp.float32),
                   jax.ShapeDtypeStruct((8, c), jnp.float32)],
        scratch_shapes=[pltpu.VMEM((K * PP, PP), jnp.float32),
                        pltpu.VMEM((8, c), jnp.float32)],
    )(y, src, b1)


def _mid_body(h_ref, st_ref, g_ref, be_ref, w_ref, b_ref,
              ho_ref, sto_ref, acc, *, c, gb, nsteps):
    t = pl.program_id(0)
    r = gb * K * PP
    hp = h_ref[...].reshape(r, c)
    mu = st_ref[0:1, :] / E_REAL
    var = st_ref[1:2, :] / E_REAL - mu * mu
    scale = g_ref[...] * lax.rsqrt(var + EPS)
    shift = be_ref[...] - mu * scale
    xn = jnp.maximum(hp * scale + shift, 0.0)
    hn = jnp.dot(xn, w_ref[...], preferred_element_type=jnp.float32) + b_ref[...]
    ho_ref[...] = hn.reshape(gb, K, PP, c)
    mrow = (lax.broadcasted_iota(jnp.int32, (1, r), 1) % PP
            < P).astype(jnp.float32)
    s = jnp.dot(mrow, hn, preferred_element_type=jnp.float32)
    ss = jnp.dot(mrow, hn * hn, preferred_element_type=jnp.float32)

    @pl.when(t == 0)
    def _():
        acc[...] = jnp.zeros_like(acc)

    acc[0:1, :] += s
    acc[1:2, :] += ss

    @pl.when(t == nsteps - 1)
    def _():
        sto_ref[...] = acc[...]


def _mid(h, st, gamma, beta, w, b, c, gb=5):
    nsteps = G // gb
    return _pc(
        functools.partial(_mid_body, c=c, gb=gb, nsteps=nsteps),
        grid=(nsteps,),
        in_specs=[pl.BlockSpec((gb, K, PP, c), lambda t: (t, 0, 0, 0)),
                  pl.BlockSpec((8, c), lambda t: (0, 0)),
                  pl.BlockSpec((1, c), lambda t: (0, 0)),
                  pl.BlockSpec((1, c), lambda t: (0, 0)),
                  pl.BlockSpec((c, c), lambda t: (0, 0)),
                  pl.BlockSpec((1, c), lambda t: (0, 0))],
        out_specs=[pl.BlockSpec((gb, K, PP, c), lambda t: (t, 0, 0, 0)),
                   pl.BlockSpec((8, c), lambda t: (0, 0))],
        out_shape=[jax.ShapeDtypeStruct((G, K, PP, c), jnp.float32),
                   jax.ShapeDtypeStruct((8, c), jnp.float32)],
        scratch_shapes=[pltpu.VMEM((8, c), jnp.float32)],
    )(h, st, gamma, beta, w, b)


def _agg_body(h_ref, st_ref, g_ref, be_ref, o_ref, *, c, gb):
    mu = st_ref[0:1, :] / E_REAL
    var = st_ref[1:2, :] / E_REAL - mu * mu
    scale = g_ref[...] * lax.rsqrt(var + EPS)
    shift = be_ref[...] - mu * scale
    h = h_ref[...]
    r = jnp.maximum(h * scale[None, None] + shift[None, None], 0.0)
    o_ref[...] = jnp.sum(r, axis=1) * (1.0 / K)


def _agg(h, st, gamma, beta, c, gb=5):
    nsteps = G // gb
    return _pc(
        functools.partial(_agg_body, c=c, gb=gb),
        grid=(nsteps,),
        in_specs=[pl.BlockSpec((gb, K, PP, c), lambda t: (t, 0, 0, 0)),
                  pl.BlockSpec((8, c), lambda t: (0, 0)),
                  pl.BlockSpec((1, c), lambda t: (0, 0)),
                  pl.BlockSpec((1, c), lambda t: (0, 0))],
        out_specs=pl.BlockSpec((gb, PP, c), lambda t: (t, 0, 0)),
        out_shape=jax.ShapeDtypeStruct((G, PP, c), jnp.float32),
    )(h, st, gamma, beta)


def _head_body(f_ref, w1_ref, b1_ref, w2_ref, b2_ref, o_ref, *, d):
    f = f_ref[...]
    real = lax.broadcasted_iota(jnp.int32, (1, PP, 1), 1) < P
    pooled = jnp.sum(jnp.where(real, f, 0.0), axis=1) * (1.0 / P)
    h = jnp.maximum(jnp.dot(pooled, w1_ref[...],
                            preferred_element_type=jnp.float32) + b1_ref[...],
                    0.0)
    z = jnp.sum(h * w2_ref[...], axis=1, keepdims=True) + b2_ref[...]
    o_ref[...] = 1.0 / (1.0 + jnp.exp(-z))


def _head(feats_p, w1, b1, w2, b2):
    d = feats_p.shape[-1]
    return _pc(
        functools.partial(_head_body, d=d),
        grid=(1,),
        in_specs=[pl.BlockSpec((G, PP, d), lambda i: (0, 0, 0)),
                  pl.BlockSpec((d, 256), lambda i: (0, 0)),
                  pl.BlockSpec((1, 256), lambda i: (0, 0)),
                  pl.BlockSpec((1, 256), lambda i: (0, 0)),
                  pl.BlockSpec((1, 1), lambda i: (0, 0))],
        out_specs=pl.BlockSpec((G, 1), lambda i: (0, 0)),
        out_shape=jax.ShapeDtypeStruct((G, 1), jnp.float32),
    )(feats_p, w1, b1, w2, b2)


def kernel(x, y, batch, params):
    xp = jnp.pad(x.reshape(G, P, -1), ((0, 0), (0, PP - P), (0, 0)))
    feats = None
    pos_dims = [2, 71, 199]
    for i in range(3):
        c = [64, 128, 256][i]
        layers = params["blocks"][i]
        if i == 0:
            src, feats = _knn(xp, pos_dims[0], pre=True)
        else:
            src = _knn(feats, pos_dims[i])
        d = feats.shape[-1]
        w0, b0, g0, be0 = layers[0]
        y1 = _nodelin(feats, jnp.concatenate([w0[:d], w0[d:]], axis=1))
        h, st = _edge1(y1, src, b0.reshape(1, c), c)
        for j in (1, 2):
            wj, bj, gj, bej = layers[j]
            gprev = layers[j - 1][2].reshape(1, c)
            beprev = layers[j - 1][3].reshape(1, c)
            h, st = _mid(h, st, gprev, beprev, wj, bj.reshape(1, c), c)
        out = _agg(h, st, layers[2][2].reshape(1, c), layers[2][3].reshape(1, c), c)
        feats = jnp.concatenate([out, feats], axis=2)
    pred = _head(feats, params["W1"], params["b1"].reshape(1, 256),
                 params["W2"].reshape(1, 256), params["b2"].reshape(1, 1))
    return (pred, y)
